# baseline (device time: 90768 ns/iter reference)
import jax
import jax.numpy as jnp
from jax import lax
from jax.experimental import pallas as pl
from jax.experimental.pallas import tpu as pltpu

N_DEV = 4


def kernel(x, w_mat):
    m, _ = x.shape
    _, n = w_mat.shape
    m_per = m // N_DEV

    def body(x_ref, w_ref, out_ref, send_buf, recv_buf, send_sems, recv_sems):
        my = lax.axis_index("i")
        right = lax.rem(my + 1, N_DEV)

        w = w_ref[...].astype(jnp.bfloat16)

        def block_partial(c):
            xb = x_ref[pl.ds(c * m_per, m_per), :].astype(jnp.bfloat16)
            return lax.dot_general(
                xb, w, (((1,), (0,)), ((), ())),
                preferred_element_type=jnp.float32,
            )

        for s in range(N_DEV - 1):
            c = lax.rem(my + (2 * N_DEV - 1 - s), N_DEV)
            part = block_partial(c)
            if s > 0:
                part = part + recv_buf[s - 1].astype(jnp.float32)
            send_buf[s, :, :] = part.astype(jnp.bfloat16)
            rdma = pltpu.make_async_remote_copy(
                src_ref=send_buf.at[s],
                dst_ref=recv_buf.at[s],
                send_sem=send_sems.at[s],
                recv_sem=recv_sems.at[s],
                device_id=(right,),
                device_id_type=pl.DeviceIdType.MESH,
            )
            rdma.start()
            rdma.wait()

        acc = block_partial(my) + recv_buf[N_DEV - 2].astype(jnp.float32)
        cg = 0.7978845608028654
        out_ref[...] = 0.5 * acc * (
            1.0 + jnp.tanh(cg * (acc + 0.044715 * acc * acc * acc))
        )

    return pl.pallas_call(
        body,
        out_shape=jax.ShapeDtypeStruct((m_per, n), jnp.float32),
        in_specs=[
            pl.BlockSpec(memory_space=pltpu.VMEM),
            pl.BlockSpec(memory_space=pltpu.VMEM),
        ],
        out_specs=pl.BlockSpec(memory_space=pltpu.VMEM),
        scratch_shapes=[
            pltpu.VMEM((N_DEV - 1, m_per, n), jnp.bfloat16),
            pltpu.VMEM((N_DEV - 1, m_per, n), jnp.bfloat16),
            pltpu.SemaphoreType.DMA((N_DEV - 1,)),
            pltpu.SemaphoreType.DMA((N_DEV - 1,)),
        ],
    )(x, w_mat)


# device time: 54463 ns/iter; 1.6666x vs baseline; 1.6666x over previous
import jax
import jax.numpy as jnp
from jax import lax
from jax.experimental import pallas as pl
from jax.experimental.pallas import tpu as pltpu

N_DEV = 4


def kernel(x, w_mat):
    m, _ = x.shape
    _, n = w_mat.shape
    m_per = m // N_DEV
    nh = n // 2

    def body(
        x_ref, w_ref, out_ref,
        part_cw, part_ccw, part_own,
        send_cw, recv_cw, send_ccw, recv_ccw,
        send_sems_cw, recv_sems_cw, send_sems_ccw, recv_sems_ccw,
    ):
        my = lax.axis_index("i")
        right = lax.rem(my + 1, N_DEV)
        left = lax.rem(my + 3, N_DEV)

        w = w_ref[...].astype(jnp.bfloat16)

        def dot_half(c, half):
            xb = x_ref[pl.ds(c * m_per, m_per), :].astype(jnp.bfloat16)
            return lax.dot_general(
                xb, w[:, half * nh:(half + 1) * nh], (((1,), (0,)), ((), ())),
                preferred_element_type=jnp.float32,
            )

        def start(src, dst, ssem, rsem, dev):
            rdma = pltpu.make_async_remote_copy(
                src_ref=src, dst_ref=dst, send_sem=ssem, recv_sem=rsem,
                device_id=(dev,), device_id_type=pl.DeviceIdType.MESH,
            )
            rdma.start()
            return rdma

        def c_cw(s):
            return lax.rem(my + (2 * N_DEV - 1 - s), N_DEV)

        def c_ccw(s):
            return lax.rem(my + 1 + s, N_DEV)

        rdmas = []

        send_cw[0, :, :] = dot_half(c_cw(0), 0).astype(jnp.bfloat16)
        rdmas.append(start(send_cw.at[0], recv_cw.at[0],
                           send_sems_cw.at[0], recv_sems_cw.at[0], right))
        send_ccw[0, :, :] = dot_half(c_ccw(0), 1).astype(jnp.bfloat16)
        rdmas.append(start(send_ccw.at[0], recv_ccw.at[0],
                           send_sems_ccw.at[0], recv_sems_ccw.at[0], left))

        for s in (1, 2):
            part_cw[s - 1, :, :] = dot_half(c_cw(s), 0)
            part_ccw[s - 1, :, :] = dot_half(c_ccw(s), 1)
        part_own[:, 0:nh] = dot_half(my, 0)
        part_own[:, nh:n] = dot_half(my, 1)

        for s in (1, 2):
            recv_rdma_cw = pltpu.make_async_remote_copy(
                src_ref=send_cw.at[s - 1], dst_ref=recv_cw.at[s - 1],
                send_sem=send_sems_cw.at[s - 1],
                recv_sem=recv_sems_cw.at[s - 1],
                device_id=(right,), device_id_type=pl.DeviceIdType.MESH,
            )
            recv_rdma_cw.wait_recv()
            send_cw[s, :, :] = (
                part_cw[s - 1] + recv_cw[s - 1].astype(jnp.float32)
            ).astype(jnp.bfloat16)
            rdmas.append(start(send_cw.at[s], recv_cw.at[s],
                               send_sems_cw.at[s], recv_sems_cw.at[s], right))

            recv_rdma_ccw = pltpu.make_async_remote_copy(
                src_ref=send_ccw.at[s - 1], dst_ref=recv_ccw.at[s - 1],
                send_sem=send_sems_ccw.at[s - 1],
                recv_sem=recv_sems_ccw.at[s - 1],
                device_id=(left,), device_id_type=pl.DeviceIdType.MESH,
            )
            recv_rdma_ccw.wait_recv()
            send_ccw[s, :, :] = (
                part_ccw[s - 1] + recv_ccw[s - 1].astype(jnp.float32)
            ).astype(jnp.bfloat16)
            rdmas.append(start(send_ccw.at[s], recv_ccw.at[s],
                               send_sems_ccw.at[s], recv_sems_ccw.at[s], left))

        last_cw = pltpu.make_async_remote_copy(
            src_ref=send_cw.at[2], dst_ref=recv_cw.at[2],
            send_sem=send_sems_cw.at[2], recv_sem=recv_sems_cw.at[2],
            device_id=(right,), device_id_type=pl.DeviceIdType.MESH,
        )
        last_cw.wait_recv()
        last_ccw = pltpu.make_async_remote_copy(
            src_ref=send_ccw.at[2], dst_ref=recv_ccw.at[2],
            send_sem=send_sems_ccw.at[2], recv_sem=recv_sems_ccw.at[2],
            device_id=(left,), device_id_type=pl.DeviceIdType.MESH,
        )
        last_ccw.wait_recv()

        cg = 0.7978845608028654

        def gelu(y):
            return 0.5 * y * (1.0 + jnp.tanh(cg * (y + 0.044715 * y * y * y)))

        acc_l = part_own[:, 0:nh] + recv_cw[2].astype(jnp.float32)
        out_ref[:, 0:nh] = gelu(acc_l)
        acc_r = part_own[:, nh:n] + recv_ccw[2].astype(jnp.float32)
        out_ref[:, nh:n] = gelu(acc_r)

        for rdma in rdmas:
            rdma.wait_send()

    return pl.pallas_call(
        body,
        out_shape=jax.ShapeDtypeStruct((m_per, n), jnp.float32),
        in_specs=[
            pl.BlockSpec(memory_space=pltpu.VMEM),
            pl.BlockSpec(memory_space=pltpu.VMEM),
        ],
        out_specs=pl.BlockSpec(memory_space=pltpu.VMEM),
        scratch_shapes=[
            pltpu.VMEM((2, m_per, nh), jnp.float32),
            pltpu.VMEM((2, m_per, nh), jnp.float32),
            pltpu.VMEM((m_per, n), jnp.float32),
            pltpu.VMEM((3, m_per, nh), jnp.bfloat16),
            pltpu.VMEM((3, m_per, nh), jnp.bfloat16),
            pltpu.VMEM((3, m_per, nh), jnp.bfloat16),
            pltpu.VMEM((3, m_per, nh), jnp.bfloat16),
            pltpu.SemaphoreType.DMA((3,)),
            pltpu.SemaphoreType.DMA((3,)),
            pltpu.SemaphoreType.DMA((3,)),
            pltpu.SemaphoreType.DMA((3,)),
        ],
    )(x, w_mat)


# device time: 47033 ns/iter; 1.9299x vs baseline; 1.1580x over previous
import jax
import jax.numpy as jnp
from jax import lax
from jax.experimental import pallas as pl
from jax.experimental.pallas import tpu as pltpu

N_DEV = 4
SUB = 4


def kernel(x, w_mat):
    m, _ = x.shape
    _, n = w_mat.shape
    m_per = m // N_DEV
    nh = n // 2
    subw = nh // SUB

    def body(
        x_ref, w_ref, out_ref,
        part_cw, part_ccw, part_own,
        send_cw, recv_cw, send_ccw, recv_ccw,
        send_sems_cw, recv_sems_cw, send_sems_ccw, recv_sems_ccw,
    ):
        my = lax.axis_index("i")
        right = lax.rem(my + 1, N_DEV)
        left = lax.rem(my + 3, N_DEV)

        barrier_sem = pltpu.get_barrier_semaphore()
        for nbr in (left, right):
            pl.semaphore_signal(
                barrier_sem, inc=1,
                device_id=(nbr,), device_id_type=pl.DeviceIdType.MESH,
            )
        pl.semaphore_wait(barrier_sem, 2)

        w = w_ref[...].astype(jnp.bfloat16)

        def dot_half(c, half):
            xb = x_ref[pl.ds(c * m_per, m_per), :].astype(jnp.bfloat16)
            return lax.dot_general(
                xb, w[:, half * nh:(half + 1) * nh], (((1,), (0,)), ((), ())),
                preferred_element_type=jnp.float32,
            )

        def make_rdma(bufs, sems, s, k, dev):
            send_buf, recv_buf = bufs
            send_sems, recv_sems = sems
            return pltpu.make_async_remote_copy(
                src_ref=send_buf.at[s, k], dst_ref=recv_buf.at[s, k],
                send_sem=send_sems.at[s, k], recv_sem=recv_sems.at[s, k],
                device_id=(dev,), device_id_type=pl.DeviceIdType.MESH,
            )

        cw_bufs = (send_cw, recv_cw)
        cw_sems = (send_sems_cw, recv_sems_cw)
        ccw_bufs = (send_ccw, recv_ccw)
        ccw_sems = (send_sems_ccw, recv_sems_ccw)

        def c_cw(s):
            return lax.rem(my + (2 * N_DEV - 1 - s), N_DEV)

        def c_ccw(s):
            return lax.rem(my + 1 + s, N_DEV)

        rdmas = []

        def cols(k):
            return pl.ds(k * subw, subw)

        h0_cw = dot_half(c_cw(0), 0)
        for k in range(SUB):
            send_cw[0, k] = h0_cw[:, k * subw:(k + 1) * subw].astype(jnp.bfloat16)
            r = make_rdma(cw_bufs, cw_sems, 0, k, right)
            r.start()
            rdmas.append(r)
        h0_ccw = dot_half(c_ccw(0), 1)
        for k in range(SUB):
            send_ccw[0, k] = h0_ccw[:, k * subw:(k + 1) * subw].astype(jnp.bfloat16)
            r = make_rdma(ccw_bufs, ccw_sems, 0, k, left)
            r.start()
            rdmas.append(r)

        for s in (1, 2):
            part_cw[s - 1] = dot_half(c_cw(s), 0)
            part_ccw[s - 1] = dot_half(c_ccw(s), 1)
        part_own[:, 0:nh] = dot_half(my, 0)
        part_own[:, nh:n] = dot_half(my, 1)

        for s in (1, 2):
            for k in range(SUB):
                make_rdma(cw_bufs, cw_sems, s - 1, k, right).wait_recv()
                send_cw[s, k] = (
                    part_cw[s - 1, :, cols(k)]
                    + recv_cw[s - 1, k].astype(jnp.float32)
                ).astype(jnp.bfloat16)
                r = make_rdma(cw_bufs, cw_sems, s, k, right)
                r.start()
                rdmas.append(r)

                make_rdma(ccw_bufs, ccw_sems, s - 1, k, left).wait_recv()
                send_ccw[s, k] = (
                    part_ccw[s - 1, :, cols(k)]
                    + recv_ccw[s - 1, k].astype(jnp.float32)
                ).astype(jnp.bfloat16)
                r = make_rdma(ccw_bufs, ccw_sems, s, k, left)
                r.start()
                rdmas.append(r)

        cg = 0.7978845608028654

        def gelu(y):
            return 0.5 * y * (1.0 + jnp.tanh(cg * (y + 0.044715 * y * y * y)))

        for k in range(SUB):
            make_rdma(cw_bufs, cw_sems, 2, k, right).wait_recv()
            acc = part_own[:, cols(k)] + recv_cw[2, k].astype(jnp.float32)
            out_ref[:, cols(k)] = gelu(acc)

            make_rdma(ccw_bufs, ccw_sems, 2, k, left).wait_recv()
            acc = (
                part_own[:, pl.ds(nh + k * subw, subw)]
                + recv_ccw[2, k].astype(jnp.float32)
            )
            out_ref[:, pl.ds(nh + k * subw, subw)] = gelu(acc)

        for r in rdmas:
            r.wait_send()

    return pl.pallas_call(
        body,
        out_shape=jax.ShapeDtypeStruct((m_per, n), jnp.float32),
        in_specs=[
            pl.BlockSpec(memory_space=pltpu.VMEM),
            pl.BlockSpec(memory_space=pltpu.VMEM),
        ],
        out_specs=pl.BlockSpec(memory_space=pltpu.VMEM),
        scratch_shapes=[
            pltpu.VMEM((2, m_per, nh), jnp.float32),
            pltpu.VMEM((2, m_per, nh), jnp.float32),
            pltpu.VMEM((m_per, n), jnp.float32),
            pltpu.VMEM((3, SUB, m_per, subw), jnp.bfloat16),
            pltpu.VMEM((3, SUB, m_per, subw), jnp.bfloat16),
            pltpu.VMEM((3, SUB, m_per, subw), jnp.bfloat16),
            pltpu.VMEM((3, SUB, m_per, subw), jnp.bfloat16),
            pltpu.SemaphoreType.DMA((3, SUB)),
            pltpu.SemaphoreType.DMA((3, SUB)),
            pltpu.SemaphoreType.DMA((3, SUB)),
            pltpu.SemaphoreType.DMA((3, SUB)),
        ],
        compiler_params=pltpu.CompilerParams(collective_id=0),
    )(x, w_mat)
